# baseline (speedup 1.0000x reference)
import functools

import jax
import jax.numpy as jnp
from jax import lax
from jax.experimental import pallas as pl
from jax.experimental.pallas import tpu as pltpu
from jax.experimental.pallas import tpu_sc as plsc

N, DIM, K = 8192, 64, 8192
D2 = 2 * DIM

BR = 256
BC = 2048
NR = N // BR
NC = K // BC


def _bf16_rnd(x):
    return x.astype(jnp.bfloat16).astype(jnp.float32)


def _vq_body(xsq_ref, ysq_ref, z_ref, embT2_ref, idx_ref, loss_ref,
             accq_s, acci_s, accv_s):
    c = pl.program_id(0)
    r = pl.program_id(1)
    s2 = jax.lax.dot_general(
        z_ref[...], embT2_ref[...], (((1,), (0,)), ((), ())),
        preferred_element_type=jnp.float32)
    d = (xsq_ref[...] + ysq_ref[...]) - s2
    vt = jnp.min(d, axis=1, keepdims=True)
    ids = jax.lax.broadcasted_iota(jnp.int32, (BR, BC), 1) + c * BC
    it = jnp.min(jnp.where(d == vt, ids, jnp.int32(2 ** 30)),
                 axis=1, keepdims=True)
    rows = pl.ds(r * BR, BR)

    @pl.when(c == 0)
    def _init():
        accq_s[rows, :] = _bf16_rnd(vt)
        acci_s[rows, :] = it
        accv_s[rows, :] = vt

    @pl.when(c > 0)
    def _update():
        accf = accq_s[rows, :]
        acci = acci_s[rows, :]
        take = (vt < accf) | ((vt == accf) & (it < acci))
        accq_s[rows, :] = _bf16_rnd(jnp.where(take, vt, accf))
        acci_s[rows, :] = jnp.where(take, it, acci)
        accv_s[rows, :] = jnp.where(take, vt, accv_s[rows, :])

    @pl.when(c == NC - 1)
    def _finish():
        idx_ref[...] = acci_s[rows, :]
        a = accv_s[rows, :] * (1.0 / D2)
        loss_ref[...] = a + 0.25 * a


@functools.partial(jax.jit, static_argnames=("interpret",))
def _vq_argmin(z_flat, emb, interpret=False):
    x_sq = jnp.sum(z_flat ** 2, axis=1, keepdims=True)
    y_sq = jnp.sum(emb ** 2, axis=1)[None, :]
    embT2 = (2.0 * emb).T
    idx, loss = pl.pallas_call(
        _vq_body,
        grid=(NC, NR),
        in_specs=[
            pl.BlockSpec((BR, 1), lambda c, r: (r, 0)),
            pl.BlockSpec((1, BC), lambda c, r: (0, c)),
            pl.BlockSpec((BR, D2), lambda c, r: (r, 0)),
            pl.BlockSpec((D2, BC), lambda c, r: (0, c)),
        ],
        out_specs=[
            pl.BlockSpec((BR, 1), lambda c, r: (r, 0)),
            pl.BlockSpec((BR, 1), lambda c, r: (r, 0)),
        ],
        out_shape=[
            jax.ShapeDtypeStruct((N, 1), jnp.int32),
            jax.ShapeDtypeStruct((N, 1), jnp.float32),
        ],
        scratch_shapes=[
            pltpu.VMEM((N, 1), jnp.float32),
            pltpu.VMEM((N, 1), jnp.int32),
            pltpu.VMEM((N, 1), jnp.float32),
        ],
        interpret=interpret,
    )(x_sq, y_sq, z_flat, embT2)
    return idx[:, 0], loss[:, 0]


NW = 32
B_W = N // NW


CH = 128
NCH = B_W // CH


def _sc_gather_hist(emb, idx, zeros_k):
    mesh = plsc.VectorSubcoreMesh(core_axis_name="c", subcore_axis_name="s")

    @functools.partial(
        pl.kernel, mesh=mesh,
        out_type=[
            jax.ShapeDtypeStruct((N, D2), jnp.float32),
            jax.ShapeDtypeStruct((2, K), jnp.float32),
        ],
        scratch_types=[
            pltpu.VMEM((CH,), jnp.int32),
            pltpu.VMEM((CH,), jnp.int32),
            pltpu.VMEM((CH, D2), jnp.float32),
            pltpu.VMEM((CH, D2), jnp.float32),
            pltpu.VMEM((CH,), jnp.float32),
            pltpu.VMEM_SHARED((K,), jnp.float32),
            pltpu.SemaphoreType.DMA,
            pltpu.SemaphoreType.DMA,
        ])
    def k(emb_hbm, idx_hbm, zeros_hbm, zq_hbm, cnt_hbm,
          idx_v0, idx_v1, rows_v0, rows_v1, ones_v, cnt_sh, sem0, sem1):
        cid = lax.axis_index("c")
        sid = lax.axis_index("s")
        wid = sid * 2 + cid
        base = wid * B_W

        @pl.when(sid == 0)
        def _zero():
            pltpu.sync_copy(zeros_hbm, cnt_sh)

        for i in range(CH // 16):
            ones_v[pl.ds(i * 16, 16)] = jnp.full((16,), 1.0, jnp.float32)

        idx_vs = (idx_v0, idx_v1)
        rows_vs = (rows_v0, rows_v1)
        sems = (sem0, sem1)
        copies = []
        for j in range(NCH):
            iv, rv, sm = idx_vs[j % 2], rows_vs[j % 2], sems[j % 2]
            pltpu.sync_copy(idx_hbm.at[pl.ds(base + j * CH, CH)], iv)
            copies.append(pltpu.async_copy(emb_hbm.at[iv], rv, sm))
        plsc.subcore_barrier()
        for j in range(NCH):
            iv, rv = idx_vs[j % 2], rows_vs[j % 2]
            copies[j].wait()
            pltpu.sync_copy(rv, zq_hbm.at[pl.ds(base + j * CH, CH)])
            pltpu.sync_copy(ones_v, cnt_sh.at[iv], add=True)
        plsc.subcore_barrier()

        @pl.when(sid == 0)
        def _out():
            pltpu.sync_copy(cnt_sh, cnt_hbm.at[cid])

    return k(emb, idx, zeros_k)


def _ent_body(cnt_ref, out_ref):
    ct = cnt_ref[0:1, :] + cnt_ref[1:2, :]
    avg = ct * (1.0 / N)
    out_ref[...] = -jnp.sum(avg * jnp.log(avg + 1e-10),
                            axis=(0, 1), keepdims=True)


def _entropy(counts2):
    out = pl.pallas_call(
        _ent_body,
        out_shape=jax.ShapeDtypeStruct((1, 1), jnp.float32),
    )(counts2)
    return out[0, 0]


def kernel(z_real, z_imag, emb):
    z_flat = jnp.concatenate([z_real, z_imag], axis=-1)
    indices, loss_sample = _vq_argmin(z_flat, emb)
    zeros_k = jnp.zeros((K,), jnp.float32)
    z_q, counts = _sc_gather_hist(emb, indices, zeros_k)
    batch_entropy = _entropy(counts)
    z_q_real = z_q[:, :DIM]
    z_q_imag = z_q[:, DIM:]
    return (z_q_real, z_q_imag, loss_sample, indices, batch_entropy)


# speedup vs baseline: 1.0475x; 1.0475x over previous
import functools

import jax
import jax.numpy as jnp
from jax import lax
from jax.experimental import pallas as pl
from jax.experimental.pallas import tpu as pltpu
from jax.experimental.pallas import tpu_sc as plsc

N, DIM, K = 8192, 64, 8192
D2 = 2 * DIM

BR = 256
BC = 2048
NR = N // BR
NC = K // BC


def _bf16_rnd(x):
    return x.astype(jnp.bfloat16).astype(jnp.float32)


def _vq_body(xsq_ref, ysq_ref, z_ref, emb_ref, idx_ref, loss_ref,
             accq_s, acci_s, accv_s):
    c = pl.program_id(0)
    r = pl.program_id(1)
    s = jax.lax.dot_general(
        z_ref[...], emb_ref[...], (((1,), (1,)), ((), ())),
        preferred_element_type=jnp.float32)
    d = (xsq_ref[...] + ysq_ref[...]) - 2.0 * s
    vt = jnp.min(d, axis=1, keepdims=True)
    ids = jax.lax.broadcasted_iota(jnp.int32, (BR, BC), 1) + c * BC
    it = jnp.min(jnp.where(d == vt, ids, jnp.int32(2 ** 30)),
                 axis=1, keepdims=True)
    rows = pl.ds(r * BR, BR)

    @pl.when(c == 0)
    def _init():
        accq_s[rows, :] = _bf16_rnd(vt)
        acci_s[rows, :] = it
        accv_s[rows, :] = vt

    @pl.when(c > 0)
    def _update():
        accf = accq_s[rows, :]
        acci = acci_s[rows, :]
        take = (vt < accf) | ((vt == accf) & (it < acci))
        accq_s[rows, :] = _bf16_rnd(jnp.where(take, vt, accf))
        acci_s[rows, :] = jnp.where(take, it, acci)
        accv_s[rows, :] = jnp.where(take, vt, accv_s[rows, :])

    @pl.when(c == NC - 1)
    def _finish():
        idx_ref[...] = acci_s[rows, :]
        a = accv_s[rows, :] * (1.0 / D2)
        loss_ref[...] = a + 0.25 * a


@functools.partial(jax.jit, static_argnames=("interpret",))
def _vq_argmin(z_flat, emb, interpret=False):
    x_sq = jnp.sum(z_flat ** 2, axis=1, keepdims=True)
    y_sq = jnp.sum(emb ** 2, axis=1)[None, :]
    idx, loss = pl.pallas_call(
        _vq_body,
        grid=(NC, NR),
        in_specs=[
            pl.BlockSpec((BR, 1), lambda c, r: (r, 0)),
            pl.BlockSpec((1, BC), lambda c, r: (0, c)),
            pl.BlockSpec((BR, D2), lambda c, r: (r, 0)),
            pl.BlockSpec((BC, D2), lambda c, r: (c, 0)),
        ],
        out_specs=[
            pl.BlockSpec((BR, 1), lambda c, r: (r, 0)),
            pl.BlockSpec((BR, 1), lambda c, r: (r, 0)),
        ],
        out_shape=[
            jax.ShapeDtypeStruct((N, 1), jnp.int32),
            jax.ShapeDtypeStruct((N, 1), jnp.float32),
        ],
        scratch_shapes=[
            pltpu.VMEM((N, 1), jnp.float32),
            pltpu.VMEM((N, 1), jnp.int32),
            pltpu.VMEM((N, 1), jnp.float32),
        ],
        interpret=interpret,
    )(x_sq, y_sq, z_flat, emb)
    return idx[:, 0], loss[:, 0]


NW = 32
B_W = N // NW


CH = 128
NCH = B_W // CH


def _sc_gather_hist(emb, idx, zeros_k):
    mesh = plsc.VectorSubcoreMesh(core_axis_name="c", subcore_axis_name="s")

    @functools.partial(
        pl.kernel, mesh=mesh,
        out_type=[
            jax.ShapeDtypeStruct((N, D2), jnp.float32),
            jax.ShapeDtypeStruct((2, K), jnp.float32),
        ],
        scratch_types=[
            pltpu.VMEM((CH,), jnp.int32),
            pltpu.VMEM((CH,), jnp.int32),
            pltpu.VMEM((CH, D2), jnp.float32),
            pltpu.VMEM((CH, D2), jnp.float32),
            pltpu.VMEM((CH,), jnp.float32),
            pltpu.VMEM_SHARED((K,), jnp.float32),
            pltpu.SemaphoreType.DMA,
            pltpu.SemaphoreType.DMA,
        ])
    def k(emb_hbm, idx_hbm, zeros_hbm, zq_hbm, cnt_hbm,
          idx_v0, idx_v1, rows_v0, rows_v1, ones_v, cnt_sh, sem0, sem1):
        cid = lax.axis_index("c")
        sid = lax.axis_index("s")
        wid = sid * 2 + cid
        base = wid * B_W

        @pl.when(sid == 0)
        def _zero():
            pltpu.sync_copy(zeros_hbm, cnt_sh)

        for i in range(CH // 16):
            ones_v[pl.ds(i * 16, 16)] = jnp.full((16,), 1.0, jnp.float32)

        idx_vs = (idx_v0, idx_v1)
        rows_vs = (rows_v0, rows_v1)
        sems = (sem0, sem1)
        copies = []
        for j in range(NCH):
            iv, rv, sm = idx_vs[j % 2], rows_vs[j % 2], sems[j % 2]
            pltpu.sync_copy(idx_hbm.at[pl.ds(base + j * CH, CH)], iv)
            copies.append(pltpu.async_copy(emb_hbm.at[iv], rv, sm))
        plsc.subcore_barrier()
        for j in range(NCH):
            iv, rv = idx_vs[j % 2], rows_vs[j % 2]
            copies[j].wait()
            pltpu.sync_copy(rv, zq_hbm.at[pl.ds(base + j * CH, CH)])
            pltpu.sync_copy(ones_v, cnt_sh.at[iv], add=True)
        plsc.subcore_barrier()

        @pl.when(sid == 0)
        def _out():
            pltpu.sync_copy(cnt_sh, cnt_hbm.at[cid])

    return k(emb, idx, zeros_k)


def _ent_body(cnt_ref, out_ref):
    ct = cnt_ref[0:1, :] + cnt_ref[1:2, :]
    avg = ct * (1.0 / N)
    out_ref[...] = -jnp.sum(avg * jnp.log(avg + 1e-10),
                            axis=(0, 1), keepdims=True)


def _entropy(counts2):
    out = pl.pallas_call(
        _ent_body,
        out_shape=jax.ShapeDtypeStruct((1, 1), jnp.float32),
    )(counts2)
    return out[0, 0]


def kernel(z_real, z_imag, emb):
    z_flat = jnp.concatenate([z_real, z_imag], axis=-1)
    indices, loss_sample = _vq_argmin(z_flat, emb)
    zeros_k = jnp.zeros((K,), jnp.float32)
    z_q, counts = _sc_gather_hist(emb, indices, zeros_k)
    batch_entropy = _entropy(counts)
    z_q_real = z_q[:, :DIM]
    z_q_imag = z_q[:, DIM:]
    return (z_q_real, z_q_imag, loss_sample, indices, batch_entropy)


# speedup vs baseline: 1.2356x; 1.1796x over previous
import functools

import jax
import jax.numpy as jnp
from jax import lax
from jax.experimental import pallas as pl
from jax.experimental.pallas import tpu as pltpu
from jax.experimental.pallas import tpu_sc as plsc

N, DIM, K = 8192, 64, 8192
D2 = 2 * DIM

BR = 512
BC = 2048
NR = N // BR
NC = K // BC


def _bf16_rnd(x):
    return x.astype(jnp.bfloat16).astype(jnp.float32)


def _vq_body(xsq_ref, ysq_ref, z_ref, emb_ref, idx_ref, loss_ref,
             accq_s, acci_s, accv_s):
    c = pl.program_id(0)
    r = pl.program_id(1)
    s = jax.lax.dot_general(
        z_ref[...], emb_ref[...], (((1,), (1,)), ((), ())),
        preferred_element_type=jnp.float32)
    d = (xsq_ref[...] + ysq_ref[...]) - 2.0 * s
    vt = jnp.min(d, axis=1, keepdims=True)
    ids = jax.lax.broadcasted_iota(jnp.int32, (BR, BC), 1)
    it = jnp.min(jnp.where(d == vt, ids, jnp.int32(2 ** 30)),
                 axis=1, keepdims=True) + c * BC
    rows = pl.ds(r * BR, BR)

    @pl.when(c == 0)
    def _init():
        accq_s[rows, :] = _bf16_rnd(vt)
        acci_s[rows, :] = it
        accv_s[rows, :] = vt

    @pl.when(c > 0)
    def _update():
        accf = accq_s[rows, :]
        acci = acci_s[rows, :]
        take = (vt < accf) | ((vt == accf) & (it < acci))
        accq_s[rows, :] = _bf16_rnd(jnp.where(take, vt, accf))
        acci_s[rows, :] = jnp.where(take, it, acci)
        accv_s[rows, :] = jnp.where(take, vt, accv_s[rows, :])

    @pl.when(c == NC - 1)
    def _finish():
        idx_ref[...] = acci_s[rows, :]
        a = accv_s[rows, :] * (1.0 / D2)
        loss_ref[...] = a + 0.25 * a


@functools.partial(jax.jit, static_argnames=("interpret",))
def _vq_argmin(z_flat, emb, interpret=False):
    x_sq = jnp.sum(z_flat ** 2, axis=1, keepdims=True)
    y_sq = jnp.sum(emb ** 2, axis=1)[None, :]
    idx, loss = pl.pallas_call(
        _vq_body,
        grid=(NC, NR),
        in_specs=[
            pl.BlockSpec((BR, 1), lambda c, r: (r, 0)),
            pl.BlockSpec((1, BC), lambda c, r: (0, c)),
            pl.BlockSpec((BR, D2), lambda c, r: (r, 0)),
            pl.BlockSpec((BC, D2), lambda c, r: (c, 0)),
        ],
        out_specs=[
            pl.BlockSpec((BR, 1), lambda c, r: (r, 0)),
            pl.BlockSpec((BR, 1), lambda c, r: (r, 0)),
        ],
        out_shape=[
            jax.ShapeDtypeStruct((N, 1), jnp.int32),
            jax.ShapeDtypeStruct((N, 1), jnp.float32),
        ],
        scratch_shapes=[
            pltpu.VMEM((N, 1), jnp.float32),
            pltpu.VMEM((N, 1), jnp.int32),
            pltpu.VMEM((N, 1), jnp.float32),
        ],
        interpret=interpret,
    )(x_sq, y_sq, z_flat, emb)
    return idx[:, 0], loss[:, 0]


NW = 32
B_W = N // NW


CH = 128
NCH = B_W // CH


def _sc_gather_hist(emb, idx, zeros_k):
    mesh = plsc.VectorSubcoreMesh(core_axis_name="c", subcore_axis_name="s")

    @functools.partial(
        pl.kernel, mesh=mesh,
        out_type=[
            jax.ShapeDtypeStruct((N, D2), jnp.float32),
            jax.ShapeDtypeStruct((2, K), jnp.float32),
        ],
        scratch_types=[
            pltpu.VMEM((CH,), jnp.int32),
            pltpu.VMEM((CH,), jnp.int32),
            pltpu.VMEM((CH, D2), jnp.float32),
            pltpu.VMEM((CH, D2), jnp.float32),
            pltpu.VMEM((CH,), jnp.float32),
            pltpu.VMEM_SHARED((K,), jnp.float32),
            pltpu.SemaphoreType.DMA,
            pltpu.SemaphoreType.DMA,
        ])
    def k(emb_hbm, idx_hbm, zeros_hbm, zq_hbm, cnt_hbm,
          idx_v0, idx_v1, rows_v0, rows_v1, ones_v, cnt_sh, sem0, sem1):
        cid = lax.axis_index("c")
        sid = lax.axis_index("s")
        wid = sid * 2 + cid
        base = wid * B_W

        @pl.when(sid == 0)
        def _zero():
            pltpu.sync_copy(zeros_hbm, cnt_sh)

        for i in range(CH // 16):
            ones_v[pl.ds(i * 16, 16)] = jnp.full((16,), 1.0, jnp.float32)

        idx_vs = (idx_v0, idx_v1)
        rows_vs = (rows_v0, rows_v1)
        sems = (sem0, sem1)
        copies = []
        for j in range(NCH):
            iv, rv, sm = idx_vs[j % 2], rows_vs[j % 2], sems[j % 2]
            pltpu.sync_copy(idx_hbm.at[pl.ds(base + j * CH, CH)], iv)
            copies.append(pltpu.async_copy(emb_hbm.at[iv], rv, sm))
        plsc.subcore_barrier()
        for j in range(NCH):
            iv, rv = idx_vs[j % 2], rows_vs[j % 2]
            copies[j].wait()
            pltpu.sync_copy(rv, zq_hbm.at[pl.ds(base + j * CH, CH)])
            pltpu.sync_copy(ones_v, cnt_sh.at[iv], add=True)
        plsc.subcore_barrier()

        @pl.when(sid == 0)
        def _out():
            pltpu.sync_copy(cnt_sh, cnt_hbm.at[cid])

    return k(emb, idx, zeros_k)


def _ent_body(cnt_ref, out_ref):
    ct = cnt_ref[0:1, :] + cnt_ref[1:2, :]
    avg = ct * (1.0 / N)
    out_ref[...] = -jnp.sum(avg * jnp.log(avg + 1e-10),
                            axis=(0, 1), keepdims=True)


def _entropy(counts2):
    out = pl.pallas_call(
        _ent_body,
        out_shape=jax.ShapeDtypeStruct((1, 1), jnp.float32),
    )(counts2)
    return out[0, 0]


def kernel(z_real, z_imag, emb):
    z_flat = jnp.concatenate([z_real, z_imag], axis=-1)
    indices, loss_sample = _vq_argmin(z_flat, emb)
    zeros_k = jnp.zeros((K,), jnp.float32)
    z_q, counts = _sc_gather_hist(emb, indices, zeros_k)
    batch_entropy = _entropy(counts)
    z_q_real = z_q[:, :DIM]
    z_q_imag = z_q[:, DIM:]
    return (z_q_real, z_q_imag, loss_sample, indices, batch_entropy)
